# initial kernel scaffold (unmeasured)
import jax
import jax.numpy as jnp
from jax import lax
from jax.experimental import pallas as pl
from jax.experimental.pallas import tpu as pltpu


def kernel(
    x,
):
    def body(*refs):
        pass

    out_shape = jax.ShapeDtypeStruct(..., jnp.float32)
    return pl.pallas_call(body, out_shape=out_shape)(...)


# baseline (device time: 19522 ns/iter reference)
import jax
import jax.numpy as jnp
from jax import lax
from jax.experimental import pallas as pl
from jax.experimental.pallas import tpu as pltpu

N_DEV = 4


def _local_reduce(x, n_blk):
    m_per, n = x.shape
    blk = m_per // n_blk

    def body(x_ref, out_ref):
        j = pl.program_id(0)
        my = lax.axis_index("i")
        chunk = x_ref[...]
        row = lax.broadcasted_iota(jnp.int32, (blk, n), 0)
        grow = row + (my * m_per + j * blk)
        cmax = jnp.max(chunk, axis=0, keepdims=True)
        big = jnp.int32(2 * N_DEV * m_per)
        cidx = jnp.min(
            jnp.where(chunk == cmax, grow, big), axis=0, keepdims=True
        ).astype(jnp.float32)

        @pl.when(j == 0)
        def _():
            out_ref[0:1, :] = cmax
            out_ref[1:2, :] = cidx

        @pl.when(j > 0)
        def _():
            run_max = out_ref[0:1, :]
            run_idx = out_ref[1:2, :]
            better = cmax > run_max
            out_ref[0:1, :] = jnp.where(better, cmax, run_max)
            out_ref[1:2, :] = jnp.where(better, cidx, run_idx)

    return pl.pallas_call(
        body,
        grid=(n_blk,),
        in_specs=[pl.BlockSpec((blk, n), lambda j: (j, 0))],
        out_specs=pl.BlockSpec((2, n), lambda j: (0, 0)),
        out_shape=jax.ShapeDtypeStruct((2, n), jnp.float32),
    )(x)


def _exchange_combine(loc):
    _, n = loc.shape

    def body(loc_ref, out_ref, gather_ref, send_sems, recv_sems):
        my = lax.axis_index("i")

        barrier_sem = pltpu.get_barrier_semaphore()
        for k in range(1, N_DEV):
            pl.semaphore_signal(
                barrier_sem,
                inc=1,
                device_id=((my + k) % N_DEV,),
                device_id_type=pl.DeviceIdType.MESH,
            )
        pl.semaphore_wait(barrier_sem, N_DEV - 1)

        gather_ref[pl.ds(my, 1), :, :] = loc_ref[...].reshape(1, 2, n)

        sends = []
        for k in range(1, N_DEV):
            peer = (my + k) % N_DEV
            d = pltpu.make_async_remote_copy(
                src_ref=gather_ref.at[pl.ds(my, 1)],
                dst_ref=gather_ref.at[pl.ds(my, 1)],
                send_sem=send_sems.at[k - 1],
                recv_sem=recv_sems.at[k - 1],
                device_id=(peer,),
                device_id_type=pl.DeviceIdType.MESH,
            )
            d.start()
            sends.append(d)
        for k in range(1, N_DEV):
            src = (my - k) % N_DEV
            r = pltpu.make_async_remote_copy(
                src_ref=gather_ref.at[pl.ds(src, 1)],
                dst_ref=gather_ref.at[pl.ds(src, 1)],
                send_sem=send_sems.at[k - 1],
                recv_sem=recv_sems.at[k - 1],
                device_id=(src,),
                device_id_type=pl.DeviceIdType.MESH,
            )
            r.wait_recv()
        for d in sends:
            d.wait_send()

        vals = gather_ref[:, 0, :]
        idxs = gather_ref[:, 1, :]
        vmax = jnp.max(vals, axis=0, keepdims=True)
        big = jnp.float32(2 * N_DEV * 8192)
        imin = jnp.min(
            jnp.where(vals == vmax, idxs, big), axis=0, keepdims=True
        )
        out_ref[0:1, :] = vmax
        out_ref[1:2, :] = imin

    return pl.pallas_call(
        body,
        out_shape=jax.ShapeDtypeStruct((2, n), jnp.float32),
        in_specs=[pl.BlockSpec(memory_space=pltpu.VMEM)],
        out_specs=pl.BlockSpec(memory_space=pltpu.VMEM),
        scratch_shapes=[
            pltpu.VMEM((N_DEV, 2, n), jnp.float32),
            pltpu.SemaphoreType.DMA((N_DEV - 1,)),
            pltpu.SemaphoreType.DMA((N_DEV - 1,)),
        ],
        compiler_params=pltpu.CompilerParams(collective_id=0),
    )(loc)


def kernel(x):
    loc = _local_reduce(x, n_blk=8)
    return _exchange_combine(loc)
